# baseline (speedup 1.0000x reference)
import functools

import jax
import jax.numpy as jnp
from jax import lax
from jax.experimental import pallas as pl
from jax.experimental.pallas import tpu as pltpu
from jax.experimental.pallas import tpu_sc as plsc

D_MODEL = 64
LANES = 16
SCALE = 8.0

SLAB_W = 128
N_FULL_SLABS = 7812
VOCAB_TAIL = 999936
SLABS_PER_W = 246
K1_TRIPLES = 82

CHUNK = 512
N_TASKS = 50


def _iota16():
    return lax.iota(jnp.int32, LANES)


def _splat16(v):
    return jnp.zeros((LANES,), jnp.int32) + v


def _transpose_compute(slab_v, trows_v, nrows, iota):
    gvecs = [g * LANES + iota for g in range(D_MODEL // LANES)]

    @plsc.parallel_loop(0, nrows, unroll=4)
    def _il_loop(il):
        ilv = _splat16(il)
        for g in range(D_MODEL // LANES):
            v = plsc.load_gather(slab_v, [gvecs[g], ilv])
            trows_v[pl.ds(il * D_MODEL + g * LANES, LANES)] = v * SCALE


def _transpose_body(tt_hbm, scr_hbm, slab0, slab1, slab2, trows0, trows1,
                    trows2, in0, in1, in2, out0, out1, out2, *, nc):
    wid = lax.axis_index("s") * nc + lax.axis_index("c")
    t0 = wid * SLABS_PER_W
    t1 = jnp.minimum(t0 + SLABS_PER_W, N_FULL_SLABS)
    iota = _iota16()
    slabs = (slab0, slab1, slab2)
    trows = (trows0, trows1, trows2)
    in_sems = (in0, in1, in2)
    out_sems = (out0, out1, out2)

    def in_copy(c, b):
        return pltpu.make_async_copy(
            tt_hbm.at[:, pl.ds(c * SLAB_W, SLAB_W)], slabs[b], in_sems[b]
        )

    def out_copy(c, b):
        return pltpu.make_async_copy(
            trows[b],
            scr_hbm.at[pl.ds(c * (SLAB_W * D_MODEL), SLAB_W * D_MODEL)],
            out_sems[b],
        )

    in_copy(t0, 0).start()

    @pl.when(t0 + 1 < t1)
    def _prime2():
        in_copy(t0 + 1, 1).start()

    def triple_body(g, carry):
        for b in range(3):
            c = t0 + g * 3 + b

            @pl.when(c < t1)
            def _step():
                in_copy(c, b).wait()

                @pl.when(c + 2 < t1)
                def _next():
                    in_copy(c + 2, (b + 2) % 3).start()

                @pl.when(c - t0 >= 3)
                def _drain():
                    out_copy(c - 3, b).wait()

                _transpose_compute(slabs[b], trows[b], SLAB_W, iota)
                out_copy(c, b).start()

        return carry

    lax.fori_loop(0, K1_TRIPLES, triple_body, 0)

    out_copy(t1 - 3, 0).wait()
    out_copy(t1 - 2, 1).wait()
    out_copy(t1 - 1, 2).wait()

    @pl.when(wid == 31)
    def _tail():
        t_start = pl.multiple_of(wid * 0 + VOCAB_TAIL, 128)
        pltpu.make_async_copy(
            tt_hbm.at[:, pl.ds(t_start, 128)], slab0, in0
        ).start()
        pltpu.make_async_copy(
            tt_hbm.at[:, pl.ds(t_start, 128)], slab0, in0
        ).wait()
        _transpose_compute(slab0, trows0, 64, iota)
        pltpu.make_async_copy(
            trows0.at[pl.ds(0, 64 * D_MODEL)],
            scr_hbm.at[pl.ds(VOCAB_TAIL * D_MODEL, 64 * D_MODEL)],
            out0,
        ).start()
        pltpu.make_async_copy(
            trows0.at[pl.ds(0, 64 * D_MODEL)],
            scr_hbm.at[pl.ds(VOCAB_TAIL * D_MODEL, 64 * D_MODEL)],
            out0,
        ).wait()


def _gather_compute(rows_v, trows_v, h, iota):
    ngroups = 16
    bvecs = [h * 256 + bg * LANES + iota for bg in range(ngroups)]
    boffs = [(bg // 8) * 1024 + (bg % 8) * LANES for bg in range(ngroups)]

    @plsc.parallel_loop(0, D_MODEL, unroll=2)
    def _d_loop(d):
        dvec = _splat16(d)
        dbase = (d // 8) * 2048 + (d % 8) * 128
        for bg in range(ngroups):
            v = plsc.load_gather(rows_v, [bvecs[bg], dvec])
            trows_v[pl.ds(dbase + boffs[bg], LANES)] = v


def _gather_body(scr_hbm, idx_hbm, out_hbm, idx0, idx1, rows0, rows1,
                 trows0, trows1, isem0, isem1, rsem0, rsem1, osem0, osem1,
                 *, nc):
    wid = lax.axis_index("s") * nc + lax.axis_index("c")
    bh0 = wid * 4
    iota = _iota16()
    idxs = (idx0, idx1)
    rows = (rows0, rows1)
    trows = (trows0, trows1)
    isems = (isem0, isem1)
    rsems = (rsem0, rsem1)
    osems = (osem0, osem1)

    def idx_copy(t, b):
        return pltpu.make_async_copy(
            idx_hbm.at[pl.ds(t * 16384 + bh0 * 128, CHUNK)], idxs[b], isems[b]
        )

    def row_copy(b):
        return pltpu.make_async_copy(scr_hbm.at[idxs[b]], rows[b], rsems[b])

    def out_copies(t, h):
        obase = t * 1048576 + (bh0 + 2 * h) * 1024
        return [
            pltpu.make_async_copy(
                trows[h].at[pl.ds(dh * 2048, 2048)],
                out_hbm.at[pl.ds(obase + dh * 131072, 2048)],
                osems[h],
            )
            for dh in range(8)
        ]

    idx_copy(0, 0).start()
    idx_copy(0, 0).wait()
    row_copy(0).start()
    idx_copy(1, 1).start()

    def task_body(ti, carry):
        for b in range(2):
            t = ti * 2 + b

            @pl.when(t + 1 < N_TASKS)
            def _next_gather():
                idx_copy(t + 1, 1 - b).wait()
                row_copy(1 - b).start()

            row_copy(b).wait()

            @pl.when(t + 2 < N_TASKS)
            def _next_idx():
                idx_copy(t + 2, b).start()

            for h in range(2):
                @pl.when(t >= 1)
                def _drain():
                    for cp in out_copies(t - 1, h):
                        cp.wait()

                _gather_compute(rows[b], trows[h], h, iota)
                for cp in out_copies(t, h):
                    cp.start()

        return carry

    lax.fori_loop(0, N_TASKS // 2, task_body, 0)

    for h in range(2):
        for cp in out_copies(N_TASKS - 1, h):
            cp.wait()


def kernel(x, table):
    b, s = x.shape
    n = b * s
    idx_t = jnp.transpose(x).reshape(n).astype(jnp.int32)
    tt = jnp.transpose(table)

    info = plsc.get_sparse_core_info()
    nc = info.num_cores
    mesh = plsc.VectorSubcoreMesh(core_axis_name="c", subcore_axis_name="s")
    params = pltpu.CompilerParams(
        use_tc_tiling_on_sc=True,
        disable_bounds_checks=True,
        needs_layout_passes=False,
    )
    params_linear = pltpu.CompilerParams(
        use_tc_tiling_on_sc=False,
        needs_layout_passes=False,
    )

    transpose_k = functools.partial(
        pl.kernel,
        mesh=mesh,
        out_type=jax.ShapeDtypeStruct((1000000 * D_MODEL,), jnp.float32),
        scratch_types=[
            pltpu.VMEM((D_MODEL, SLAB_W), jnp.float32),
            pltpu.VMEM((D_MODEL, SLAB_W), jnp.float32),
            pltpu.VMEM((D_MODEL, SLAB_W), jnp.float32),
            pltpu.VMEM((SLAB_W * D_MODEL,), jnp.float32),
            pltpu.VMEM((SLAB_W * D_MODEL,), jnp.float32),
            pltpu.VMEM((SLAB_W * D_MODEL,), jnp.float32),
            pltpu.SemaphoreType.DMA,
            pltpu.SemaphoreType.DMA,
            pltpu.SemaphoreType.DMA,
            pltpu.SemaphoreType.DMA,
            pltpu.SemaphoreType.DMA,
            pltpu.SemaphoreType.DMA,
        ],
        compiler_params=params,
    )(functools.partial(_transpose_body, nc=nc))

    gather_k = functools.partial(
        pl.kernel,
        mesh=mesh,
        out_type=jax.ShapeDtypeStruct((50 * 8 * 128 * 8 * 128,), jnp.float32),
        scratch_types=[
            pltpu.VMEM((CHUNK,), jnp.int32),
            pltpu.VMEM((CHUNK,), jnp.int32),
            pltpu.VMEM((CHUNK, D_MODEL), jnp.float32),
            pltpu.VMEM((CHUNK, D_MODEL), jnp.float32),
            pltpu.VMEM((128 * 128,), jnp.float32),
            pltpu.VMEM((128 * 128,), jnp.float32),
            pltpu.SemaphoreType.DMA,
            pltpu.SemaphoreType.DMA,
            pltpu.SemaphoreType.DMA,
            pltpu.SemaphoreType.DMA,
            pltpu.SemaphoreType.DMA,
            pltpu.SemaphoreType.DMA,
        ],
        compiler_params=params_linear,
    )(functools.partial(_gather_body, nc=nc))

    scr = transpose_k(tt)
    out1 = gather_k(scr.reshape(1000000, D_MODEL), idx_t)
    out5 = out1.reshape(50, 8, 128, 8, 128)
    return out5.transpose(2, 4, 0, 1, 3).reshape(b, s, D_MODEL)


# speedup vs baseline: 1.4871x; 1.4871x over previous
import functools

import jax
import jax.numpy as jnp
from jax import lax
from jax.experimental import pallas as pl
from jax.experimental.pallas import tpu as pltpu
from jax.experimental.pallas import tpu_sc as plsc

D_MODEL = 64
LANES = 16
SCALE = 8.0

SLAB_W = 128
N_FULL_SLABS = 7812
VOCAB_TAIL = 999936
SLABS_PER_W = 246
K1_TRIPLES = 82

CHUNK = 512
N_TASKS = 50


def _iota16():
    return lax.iota(jnp.int32, LANES)


def _splat16(v):
    return jnp.zeros((LANES,), jnp.int32) + v


def _transpose_compute(slab_v, trows_v, nrows, iota):
    rot = [(iota + k) & 15 for k in range(LANES)]
    rot64i = [rot[k] * D_MODEL + iota for k in range(LANES)]
    gvecs = [g * LANES + iota for g in range(D_MODEL // LANES)]

    @plsc.parallel_loop(0, nrows // LANES)
    def _t_loop(t):
        tv16 = _splat16(t * LANES)
        tv1k = _splat16(t * LANES * D_MODEL)
        for g in range(D_MODEL // LANES):
            for k in range(LANES):
                cvec = tv16 + rot[k]
                v = plsc.load_gather(slab_v, [gvecs[g], cvec])
                sidx = tv1k + (rot64i[k] + g * LANES)
                plsc.store_scatter(trows_v, [sidx], v * SCALE)


def _transpose_body(tt_hbm, scr_hbm, slab0, slab1, slab2, trows0, trows1,
                    trows2, in0, in1, in2, out0, out1, out2, *, nc):
    wid = lax.axis_index("s") * nc + lax.axis_index("c")
    t0 = wid * SLABS_PER_W
    t1 = jnp.minimum(t0 + SLABS_PER_W, N_FULL_SLABS)
    iota = _iota16()
    slabs = (slab0, slab1, slab2)
    trows = (trows0, trows1, trows2)
    in_sems = (in0, in1, in2)
    out_sems = (out0, out1, out2)

    def in_copy(c, b):
        return pltpu.make_async_copy(
            tt_hbm.at[:, pl.ds(c * SLAB_W, SLAB_W)], slabs[b], in_sems[b]
        )

    def out_copy(c, b):
        return pltpu.make_async_copy(
            trows[b],
            scr_hbm.at[pl.ds(c * (SLAB_W * D_MODEL), SLAB_W * D_MODEL)],
            out_sems[b],
        )

    in_copy(t0, 0).start()

    @pl.when(t0 + 1 < t1)
    def _prime2():
        in_copy(t0 + 1, 1).start()

    def triple_body(g, carry):
        for b in range(3):
            c = t0 + g * 3 + b

            @pl.when(c < t1)
            def _step():
                in_copy(c, b).wait()

                @pl.when(c + 2 < t1)
                def _next():
                    in_copy(c + 2, (b + 2) % 3).start()

                @pl.when(c - t0 >= 3)
                def _drain():
                    out_copy(c - 3, b).wait()

                _transpose_compute(slabs[b], trows[b], SLAB_W, iota)
                out_copy(c, b).start()

        return carry

    lax.fori_loop(0, K1_TRIPLES, triple_body, 0)

    out_copy(t1 - 3, 0).wait()
    out_copy(t1 - 2, 1).wait()
    out_copy(t1 - 1, 2).wait()

    @pl.when(wid == 31)
    def _tail():
        t_start = pl.multiple_of(wid * 0 + VOCAB_TAIL, 128)
        pltpu.make_async_copy(
            tt_hbm.at[:, pl.ds(t_start, 128)], slab0, in0
        ).start()
        pltpu.make_async_copy(
            tt_hbm.at[:, pl.ds(t_start, 128)], slab0, in0
        ).wait()
        _transpose_compute(slab0, trows0, 64, iota)
        pltpu.make_async_copy(
            trows0.at[pl.ds(0, 64 * D_MODEL)],
            scr_hbm.at[pl.ds(VOCAB_TAIL * D_MODEL, 64 * D_MODEL)],
            out0,
        ).start()
        pltpu.make_async_copy(
            trows0.at[pl.ds(0, 64 * D_MODEL)],
            scr_hbm.at[pl.ds(VOCAB_TAIL * D_MODEL, 64 * D_MODEL)],
            out0,
        ).wait()


def _gather_compute(rows_v, trows_v, h, iota):
    rot = [(iota + k) & 15 for k in range(LANES)]
    rot_d = [(rot[k] // 8) * 2048 + (rot[k] % 8) * 128 + iota
             for k in range(LANES)]
    biota = h * 256 + iota

    @plsc.parallel_loop(0, 16)
    def _blk_loop(blk):
        bvec = biota + blk * LANES
        b_store = _splat16(((blk * LANES) // 128) * 1024 + (blk * LANES) % 128)
        for g in range(D_MODEL // LANES):
            for k in range(LANES):
                dvec = rot[k] + g * LANES
                v = plsc.load_gather(rows_v, [bvec, dvec])
                sidx = b_store + (rot_d[k] + g * 4096)
                plsc.store_scatter(trows_v, [sidx], v)


def _gather_body(scr_hbm, idx_hbm, out_hbm, idx0, idx1, rows0, rows1,
                 trows0, trows1, isem0, isem1, rsem0, rsem1, osem0, osem1,
                 *, nc):
    wid = lax.axis_index("s") * nc + lax.axis_index("c")
    bh0 = wid * 4
    iota = _iota16()
    idxs = (idx0, idx1)
    rows = (rows0, rows1)
    trows = (trows0, trows1)
    isems = (isem0, isem1)
    rsems = (rsem0, rsem1)
    osems = (osem0, osem1)

    def idx_copy(t, b):
        return pltpu.make_async_copy(
            idx_hbm.at[pl.ds(t * 16384 + bh0 * 128, CHUNK)], idxs[b], isems[b]
        )

    def row_copy(b):
        return pltpu.make_async_copy(scr_hbm.at[idxs[b]], rows[b], rsems[b])

    def out_copies(t, h):
        obase = t * 1048576 + (bh0 + 2 * h) * 1024
        return [
            pltpu.make_async_copy(
                trows[h].at[pl.ds(dh * 2048, 2048)],
                out_hbm.at[pl.ds(obase + dh * 131072, 2048)],
                osems[h],
            )
            for dh in range(8)
        ]

    idx_copy(0, 0).start()
    idx_copy(0, 0).wait()
    row_copy(0).start()
    idx_copy(1, 1).start()

    def task_body(ti, carry):
        for b in range(2):
            t = ti * 2 + b

            @pl.when(t + 1 < N_TASKS)
            def _next_gather():
                idx_copy(t + 1, 1 - b).wait()
                row_copy(1 - b).start()

            row_copy(b).wait()

            @pl.when(t + 2 < N_TASKS)
            def _next_idx():
                idx_copy(t + 2, b).start()

            for h in range(2):
                @pl.when(t >= 1)
                def _drain():
                    for cp in out_copies(t - 1, h):
                        cp.wait()

                _gather_compute(rows[b], trows[h], h, iota)
                for cp in out_copies(t, h):
                    cp.start()

        return carry

    lax.fori_loop(0, N_TASKS // 2, task_body, 0)

    for h in range(2):
        for cp in out_copies(N_TASKS - 1, h):
            cp.wait()


def kernel(x, table):
    b, s = x.shape
    n = b * s
    idx_t = jnp.transpose(x).reshape(n).astype(jnp.int32)
    tt = jnp.transpose(table)

    info = plsc.get_sparse_core_info()
    nc = info.num_cores
    mesh = plsc.VectorSubcoreMesh(core_axis_name="c", subcore_axis_name="s")
    params = pltpu.CompilerParams(
        use_tc_tiling_on_sc=True,
        disable_bounds_checks=True,
        needs_layout_passes=False,
    )
    params_linear = pltpu.CompilerParams(
        use_tc_tiling_on_sc=False,
        needs_layout_passes=False,
    )

    transpose_k = functools.partial(
        pl.kernel,
        mesh=mesh,
        out_type=jax.ShapeDtypeStruct((1000000 * D_MODEL,), jnp.float32),
        scratch_types=[
            pltpu.VMEM((D_MODEL, SLAB_W), jnp.float32),
            pltpu.VMEM((D_MODEL, SLAB_W), jnp.float32),
            pltpu.VMEM((D_MODEL, SLAB_W), jnp.float32),
            pltpu.VMEM((SLAB_W * D_MODEL,), jnp.float32),
            pltpu.VMEM((SLAB_W * D_MODEL,), jnp.float32),
            pltpu.VMEM((SLAB_W * D_MODEL,), jnp.float32),
            pltpu.SemaphoreType.DMA,
            pltpu.SemaphoreType.DMA,
            pltpu.SemaphoreType.DMA,
            pltpu.SemaphoreType.DMA,
            pltpu.SemaphoreType.DMA,
            pltpu.SemaphoreType.DMA,
        ],
        compiler_params=params,
    )(functools.partial(_transpose_body, nc=nc))

    gather_k = functools.partial(
        pl.kernel,
        mesh=mesh,
        out_type=jax.ShapeDtypeStruct((50 * 8 * 128 * 8 * 128,), jnp.float32),
        scratch_types=[
            pltpu.VMEM((CHUNK,), jnp.int32),
            pltpu.VMEM((CHUNK,), jnp.int32),
            pltpu.VMEM((CHUNK, D_MODEL), jnp.float32),
            pltpu.VMEM((CHUNK, D_MODEL), jnp.float32),
            pltpu.VMEM((128 * 128,), jnp.float32),
            pltpu.VMEM((128 * 128,), jnp.float32),
            pltpu.SemaphoreType.DMA,
            pltpu.SemaphoreType.DMA,
            pltpu.SemaphoreType.DMA,
            pltpu.SemaphoreType.DMA,
            pltpu.SemaphoreType.DMA,
            pltpu.SemaphoreType.DMA,
        ],
        compiler_params=params_linear,
    )(functools.partial(_gather_body, nc=nc))

    scr = transpose_k(tt)
    out1 = gather_k(scr.reshape(1000000, D_MODEL), idx_t)
    out5 = out1.reshape(50, 8, 128, 8, 128)
    return out5.transpose(2, 4, 0, 1, 3).reshape(b, s, D_MODEL)


# speedup vs baseline: 1.5093x; 1.0149x over previous
import functools

import jax
import jax.numpy as jnp
from jax import lax
from jax.experimental import pallas as pl
from jax.experimental.pallas import tpu as pltpu
from jax.experimental.pallas import tpu_sc as plsc

D_MODEL = 64
LANES = 16
SCALE = 8.0

SLAB_W = 128
N_FULL_SLABS = 7812
VOCAB_TAIL = 999936
SLABS_PER_W = 246
K1_TRIPLES = 82

CHUNK = 512
N_TASKS = 50


def _iota16():
    return lax.iota(jnp.int32, LANES)


def _splat16(v):
    return jnp.zeros((LANES,), jnp.int32) + v


def _transpose_compute(slab_v, trows_v, nrows, iota):
    rot = [(iota + k) & 15 for k in range(LANES)]
    rot64i = [rot[k] * D_MODEL + iota for k in range(LANES)]
    gvecs = [g * LANES + iota for g in range(D_MODEL // LANES)]

    @plsc.parallel_loop(0, nrows // LANES)
    def _t_loop(t):
        tv16 = _splat16(t * LANES)
        tblock = trows_v.at[pl.ds(t * (LANES * D_MODEL), LANES * D_MODEL)]
        for k in range(LANES):
            cvec = tv16 + rot[k]
            for g in range(D_MODEL // LANES):
                v = plsc.load_gather(slab_v, [gvecs[g], cvec])
                plsc.store_scatter(tblock, [rot64i[k] + g * LANES], v * SCALE)


def _transpose_body(tt_hbm, scr_hbm, slab0, slab1, slab2, trows0, trows1,
                    trows2, in0, in1, in2, out0, out1, out2, *, nc):
    wid = lax.axis_index("s") * nc + lax.axis_index("c")
    t0 = wid * SLABS_PER_W
    t1 = jnp.minimum(t0 + SLABS_PER_W, N_FULL_SLABS)
    iota = _iota16()
    slabs = (slab0, slab1, slab2)
    trows = (trows0, trows1, trows2)
    in_sems = (in0, in1, in2)
    out_sems = (out0, out1, out2)

    def in_copy(c, b):
        return pltpu.make_async_copy(
            tt_hbm.at[:, pl.ds(c * SLAB_W, SLAB_W)], slabs[b], in_sems[b]
        )

    def out_copy(c, b):
        return pltpu.make_async_copy(
            trows[b],
            scr_hbm.at[pl.ds(c * (SLAB_W * D_MODEL), SLAB_W * D_MODEL)],
            out_sems[b],
        )

    in_copy(t0, 0).start()

    @pl.when(t0 + 1 < t1)
    def _prime2():
        in_copy(t0 + 1, 1).start()

    def triple_body(g, carry):
        for b in range(3):
            c = t0 + g * 3 + b

            @pl.when(c < t1)
            def _step():
                in_copy(c, b).wait()

                @pl.when(c + 2 < t1)
                def _next():
                    in_copy(c + 2, (b + 2) % 3).start()

                @pl.when(c - t0 >= 3)
                def _drain():
                    out_copy(c - 3, b).wait()

                _transpose_compute(slabs[b], trows[b], SLAB_W, iota)
                out_copy(c, b).start()

        return carry

    lax.fori_loop(0, K1_TRIPLES, triple_body, 0)

    out_copy(t1 - 3, 0).wait()
    out_copy(t1 - 2, 1).wait()
    out_copy(t1 - 1, 2).wait()

    @pl.when(wid == 31)
    def _tail():
        t_start = pl.multiple_of(wid * 0 + VOCAB_TAIL, 128)
        pltpu.make_async_copy(
            tt_hbm.at[:, pl.ds(t_start, 128)], slab0, in0
        ).start()
        pltpu.make_async_copy(
            tt_hbm.at[:, pl.ds(t_start, 128)], slab0, in0
        ).wait()
        _transpose_compute(slab0, trows0, 64, iota)
        pltpu.make_async_copy(
            trows0.at[pl.ds(0, 64 * D_MODEL)],
            scr_hbm.at[pl.ds(VOCAB_TAIL * D_MODEL, 64 * D_MODEL)],
            out0,
        ).start()
        pltpu.make_async_copy(
            trows0.at[pl.ds(0, 64 * D_MODEL)],
            scr_hbm.at[pl.ds(VOCAB_TAIL * D_MODEL, 64 * D_MODEL)],
            out0,
        ).wait()


def _gather_compute(rows_v, trows_v, h, iota):
    rot = [(iota + k) & 15 for k in range(LANES)]
    rot_d = [(rot[k] // 8) * 2048 + (rot[k] % 8) * 128 + iota
             for k in range(LANES)]
    biota = h * 256 + iota

    @plsc.parallel_loop(0, 16)
    def _blk_loop(blk):
        bvec = biota + blk * LANES
        b_store = _splat16(((blk * LANES) // 128) * 1024 + (blk * LANES) % 128)
        for g in range(D_MODEL // LANES):
            for k in range(LANES):
                dvec = rot[k] + g * LANES
                v = plsc.load_gather(rows_v, [bvec, dvec])
                sidx = b_store + (rot_d[k] + g * 4096)
                plsc.store_scatter(trows_v, [sidx], v)


def _gather_body(scr_hbm, idx_hbm, out_hbm, idx0, idx1, rows0, rows1,
                 trows0, trows1, isem0, isem1, rsem0, rsem1, osem0, osem1,
                 *, nc):
    wid = lax.axis_index("s") * nc + lax.axis_index("c")
    bh0 = wid * 4
    iota = _iota16()
    idxs = (idx0, idx1)
    rows = (rows0, rows1)
    trows = (trows0, trows1)
    isems = (isem0, isem1)
    rsems = (rsem0, rsem1)
    osems = (osem0, osem1)

    def idx_copy(t, b):
        return pltpu.make_async_copy(
            idx_hbm.at[pl.ds(t * 16384 + bh0 * 128, CHUNK)], idxs[b], isems[b]
        )

    def row_copy(b):
        return pltpu.make_async_copy(scr_hbm.at[idxs[b]], rows[b], rsems[b])

    def out_copies(t, h):
        obase = t * 1048576 + (bh0 + 2 * h) * 1024
        return [
            pltpu.make_async_copy(
                trows[h].at[pl.ds(dh * 2048, 2048)],
                out_hbm.at[pl.ds(obase + dh * 131072, 2048)],
                osems[h],
            )
            for dh in range(8)
        ]

    idx_copy(0, 0).start()
    idx_copy(0, 0).wait()
    row_copy(0).start()
    idx_copy(1, 1).start()

    def task_body(ti, carry):
        for b in range(2):
            t = ti * 2 + b

            @pl.when(t + 1 < N_TASKS)
            def _next_gather():
                idx_copy(t + 1, 1 - b).wait()
                row_copy(1 - b).start()

            row_copy(b).wait()

            @pl.when(t + 2 < N_TASKS)
            def _next_idx():
                idx_copy(t + 2, b).start()

            for h in range(2):
                @pl.when(t >= 1)
                def _drain():
                    for cp in out_copies(t - 1, h):
                        cp.wait()

                _gather_compute(rows[b], trows[h], h, iota)
                for cp in out_copies(t, h):
                    cp.start()

        return carry

    lax.fori_loop(0, N_TASKS // 2, task_body, 0)

    for h in range(2):
        for cp in out_copies(N_TASKS - 1, h):
            cp.wait()


def kernel(x, table):
    b, s = x.shape
    n = b * s
    idx_t = jnp.transpose(x).reshape(n).astype(jnp.int32)
    tt = jnp.transpose(table)

    info = plsc.get_sparse_core_info()
    nc = info.num_cores
    mesh = plsc.VectorSubcoreMesh(core_axis_name="c", subcore_axis_name="s")
    params = pltpu.CompilerParams(
        use_tc_tiling_on_sc=True,
        disable_bounds_checks=True,
        needs_layout_passes=False,
    )
    params_linear = pltpu.CompilerParams(
        use_tc_tiling_on_sc=False,
        needs_layout_passes=False,
    )

    transpose_k = functools.partial(
        pl.kernel,
        mesh=mesh,
        out_type=jax.ShapeDtypeStruct((1000000 * D_MODEL,), jnp.float32),
        scratch_types=[
            pltpu.VMEM((D_MODEL, SLAB_W), jnp.float32),
            pltpu.VMEM((D_MODEL, SLAB_W), jnp.float32),
            pltpu.VMEM((D_MODEL, SLAB_W), jnp.float32),
            pltpu.VMEM((SLAB_W * D_MODEL,), jnp.float32),
            pltpu.VMEM((SLAB_W * D_MODEL,), jnp.float32),
            pltpu.VMEM((SLAB_W * D_MODEL,), jnp.float32),
            pltpu.SemaphoreType.DMA,
            pltpu.SemaphoreType.DMA,
            pltpu.SemaphoreType.DMA,
            pltpu.SemaphoreType.DMA,
            pltpu.SemaphoreType.DMA,
            pltpu.SemaphoreType.DMA,
        ],
        compiler_params=params,
    )(functools.partial(_transpose_body, nc=nc))

    gather_k = functools.partial(
        pl.kernel,
        mesh=mesh,
        out_type=jax.ShapeDtypeStruct((50 * 8 * 128 * 8 * 128,), jnp.float32),
        scratch_types=[
            pltpu.VMEM((CHUNK,), jnp.int32),
            pltpu.VMEM((CHUNK,), jnp.int32),
            pltpu.VMEM((CHUNK, D_MODEL), jnp.float32),
            pltpu.VMEM((CHUNK, D_MODEL), jnp.float32),
            pltpu.VMEM((128 * 128,), jnp.float32),
            pltpu.VMEM((128 * 128,), jnp.float32),
            pltpu.SemaphoreType.DMA,
            pltpu.SemaphoreType.DMA,
            pltpu.SemaphoreType.DMA,
            pltpu.SemaphoreType.DMA,
            pltpu.SemaphoreType.DMA,
            pltpu.SemaphoreType.DMA,
        ],
        compiler_params=params_linear,
    )(functools.partial(_gather_body, nc=nc))

    scr = transpose_k(tt)
    out1 = gather_k(scr.reshape(1000000, D_MODEL), idx_t)
    out5 = out1.reshape(50, 8, 128, 8, 128)
    return out5.transpose(2, 4, 0, 1, 3).reshape(b, s, D_MODEL)


# speedup vs baseline: 1.5271x; 1.0118x over previous
import functools

import jax
import jax.numpy as jnp
from jax import lax
from jax.experimental import pallas as pl
from jax.experimental.pallas import tpu as pltpu
from jax.experimental.pallas import tpu_sc as plsc

D_MODEL = 64
LANES = 16
SCALE = 8.0

SLAB_W = 128
N_FULL_SLABS = 7812
VOCAB_TAIL = 999936
SLABS_PER_W = 246
K1_TRIPLES = 82

CHUNK = 512
N_TASKS = 50


def _iota16():
    return lax.iota(jnp.int32, LANES)


def _splat16(v):
    return jnp.zeros((LANES,), jnp.int32) + v


def _transpose_compute(slab_v, trows_v, nrows, iota):
    rot = [(iota + k) & 15 for k in range(LANES)]
    rot64i = [rot[k] * D_MODEL + iota for k in range(LANES)]
    gvecs = [g * LANES + iota for g in range(D_MODEL // LANES)]

    @plsc.parallel_loop(0, nrows // LANES)
    def _t_loop(t):
        tv16 = _splat16(t * LANES)
        tv1k = _splat16(t * LANES * D_MODEL)
        for g in range(D_MODEL // LANES):
            for k in range(LANES):
                cvec = tv16 + rot[k]
                v = plsc.load_gather(slab_v, [gvecs[g], cvec])
                sidx = tv1k + (rot64i[k] + g * LANES)
                plsc.store_scatter(trows_v, [sidx], v * SCALE)


def _transpose_body(tt_hbm, scr_hbm, slab0, slab1, slab2, trows0, trows1,
                    trows2, in0, in1, in2, out0, out1, out2, *, nc):
    wid = lax.axis_index("s") * nc + lax.axis_index("c")
    t0 = wid * SLABS_PER_W
    t1 = jnp.minimum(t0 + SLABS_PER_W, N_FULL_SLABS)
    iota = _iota16()
    slabs = (slab0, slab1, slab2)
    trows = (trows0, trows1, trows2)
    in_sems = (in0, in1, in2)
    out_sems = (out0, out1, out2)

    def in_copy(c, b):
        return pltpu.make_async_copy(
            tt_hbm.at[:, pl.ds(c * SLAB_W, SLAB_W)], slabs[b], in_sems[b]
        )

    def out_copy(c, b):
        return pltpu.make_async_copy(
            trows[b],
            scr_hbm.at[pl.ds(c * (SLAB_W * D_MODEL), SLAB_W * D_MODEL)],
            out_sems[b],
        )

    in_copy(t0, 0).start()

    @pl.when(t0 + 1 < t1)
    def _prime2():
        in_copy(t0 + 1, 1).start()

    def triple_body(g, carry):
        for b in range(3):
            c = t0 + g * 3 + b

            @pl.when(c < t1)
            def _step():
                in_copy(c, b).wait()

                @pl.when(c + 2 < t1)
                def _next():
                    in_copy(c + 2, (b + 2) % 3).start()

                @pl.when(c - t0 >= 3)
                def _drain():
                    out_copy(c - 3, b).wait()

                _transpose_compute(slabs[b], trows[b], SLAB_W, iota)
                out_copy(c, b).start()

        return carry

    lax.fori_loop(0, K1_TRIPLES, triple_body, 0)

    out_copy(t1 - 3, 0).wait()
    out_copy(t1 - 2, 1).wait()
    out_copy(t1 - 1, 2).wait()

    @pl.when(wid == 31)
    def _tail():
        t_start = pl.multiple_of(wid * 0 + VOCAB_TAIL, 128)
        pltpu.make_async_copy(
            tt_hbm.at[:, pl.ds(t_start, 128)], slab0, in0
        ).start()
        pltpu.make_async_copy(
            tt_hbm.at[:, pl.ds(t_start, 128)], slab0, in0
        ).wait()
        _transpose_compute(slab0, trows0, 64, iota)
        pltpu.make_async_copy(
            trows0.at[pl.ds(0, 64 * D_MODEL)],
            scr_hbm.at[pl.ds(VOCAB_TAIL * D_MODEL, 64 * D_MODEL)],
            out0,
        ).start()
        pltpu.make_async_copy(
            trows0.at[pl.ds(0, 64 * D_MODEL)],
            scr_hbm.at[pl.ds(VOCAB_TAIL * D_MODEL, 64 * D_MODEL)],
            out0,
        ).wait()


def _gather_compute(rows_v, trows_v, h, iota):
    rot = [(iota + k) & 15 for k in range(LANES)]
    rot_d = [(rot[k] // 8) * 2048 + (rot[k] % 8) * 128 + iota
             for k in range(LANES)]
    biota = h * 256 + iota

    @plsc.parallel_loop(0, 16)
    def _blk_loop(blk):
        bvec = biota + blk * LANES
        b_store = _splat16(((blk * LANES) // 128) * 1024 + (blk * LANES) % 128)
        for g in range(D_MODEL // LANES):
            for k in range(LANES):
                dvec = rot[k] + g * LANES
                v = plsc.load_gather(rows_v, [bvec, dvec])
                sidx = b_store + (rot_d[k] + g * 4096)
                plsc.store_scatter(trows_v, [sidx], v)


def _gather_body(scr_hbm, idx_hbm, out_hbm, idx0, idx1, rows0, rows1,
                 trows0, trows1, isem0, isem1, rsem0, rsem1, osem0, osem1,
                 *, nc):
    wid = lax.axis_index("s") * nc + lax.axis_index("c")
    bh0 = wid * 4
    iota = _iota16()
    idxs = (idx0, idx1)
    rows = (rows0, rows1)
    trows = (trows0, trows1)
    isems = (isem0, isem1)
    rsems = (rsem0, rsem1)
    osems = (osem0, osem1)

    def idx_copy(t, b):
        return pltpu.make_async_copy(
            idx_hbm.at[pl.ds(t * 16384 + bh0 * 128, CHUNK)], idxs[b], isems[b]
        )

    def row_copy(b):
        return pltpu.make_async_copy(scr_hbm.at[idxs[b]], rows[b], rsems[b])

    def out_copies(t, h):
        obase = t * 1048576 + (bh0 + 2 * h) * 1024
        return [
            pltpu.make_async_copy(
                trows[h].at[pl.ds(dh * 2048, 2048)],
                out_hbm.at[pl.ds(obase + dh * 131072, 2048)],
                osems[h],
            )
            for dh in range(8)
        ]

    idx_copy(0, 0).start()
    idx_copy(0, 0).wait()
    row_copy(0).start()
    idx_copy(1, 1).start()

    def task_body(ti, carry):
        for b in range(2):
            t = ti * 2 + b

            @pl.when(t + 1 < N_TASKS)
            def _next_gather():
                idx_copy(t + 1, 1 - b).wait()
                row_copy(1 - b).start()

            row_copy(b).wait()

            @pl.when(t + 2 < N_TASKS)
            def _next_idx():
                idx_copy(t + 2, b).start()

            for h in range(2):
                @pl.when(t >= 1)
                def _drain():
                    for cp in out_copies(t - 1, h):
                        cp.wait()

                _gather_compute(rows[b], trows[h], h, iota)
                for cp in out_copies(t, h):
                    cp.start()

        return carry

    lax.fori_loop(0, N_TASKS // 2, task_body, 0)

    for h in range(2):
        for cp in out_copies(N_TASKS - 1, h):
            cp.wait()


def kernel(x, table):
    b, s = x.shape
    n = b * s
    idx_t = jnp.transpose(x).reshape(n).astype(jnp.int32)
    tt = jnp.transpose(table)

    info = plsc.get_sparse_core_info()
    nc = info.num_cores
    mesh = plsc.VectorSubcoreMesh(core_axis_name="c", subcore_axis_name="s")
    params = pltpu.CompilerParams(
        use_tc_tiling_on_sc=True,
        disable_bounds_checks=True,
        needs_layout_passes=False,
    )
    params_linear = pltpu.CompilerParams(
        use_tc_tiling_on_sc=False,
        needs_layout_passes=False,
    )

    transpose_k = functools.partial(
        pl.kernel,
        mesh=mesh,
        out_type=jax.ShapeDtypeStruct((1000000 * D_MODEL,), jnp.float32),
        scratch_types=[
            pltpu.VMEM((D_MODEL, SLAB_W), jnp.float32),
            pltpu.VMEM((D_MODEL, SLAB_W), jnp.float32),
            pltpu.VMEM((D_MODEL, SLAB_W), jnp.float32),
            pltpu.VMEM((SLAB_W * D_MODEL,), jnp.float32),
            pltpu.VMEM((SLAB_W * D_MODEL,), jnp.float32),
            pltpu.VMEM((SLAB_W * D_MODEL,), jnp.float32),
            pltpu.SemaphoreType.DMA,
            pltpu.SemaphoreType.DMA,
            pltpu.SemaphoreType.DMA,
            pltpu.SemaphoreType.DMA,
            pltpu.SemaphoreType.DMA,
            pltpu.SemaphoreType.DMA,
        ],
        compiler_params=params,
    )(functools.partial(_transpose_body, nc=nc))

    gather_k = functools.partial(
        pl.kernel,
        mesh=mesh,
        out_type=jax.ShapeDtypeStruct((50 * 8 * 128 * 8 * 128,), jnp.float32),
        scratch_types=[
            pltpu.VMEM((CHUNK,), jnp.int32),
            pltpu.VMEM((CHUNK,), jnp.int32),
            pltpu.VMEM((CHUNK, D_MODEL), jnp.float32),
            pltpu.VMEM((CHUNK, D_MODEL), jnp.float32),
            pltpu.VMEM((128 * 128,), jnp.float32),
            pltpu.VMEM((128 * 128,), jnp.float32),
            pltpu.SemaphoreType.DMA,
            pltpu.SemaphoreType.DMA,
            pltpu.SemaphoreType.DMA,
            pltpu.SemaphoreType.DMA,
            pltpu.SemaphoreType.DMA,
            pltpu.SemaphoreType.DMA,
        ],
        compiler_params=params_linear,
    )(functools.partial(_gather_body, nc=nc))

    scr = transpose_k(tt)
    out1 = gather_k(scr.reshape(1000000, D_MODEL), idx_t)
    out5 = out1.reshape(50, 8, 128, 8, 128)
    return out5.transpose(2, 4, 0, 1, 3).reshape(b, s, D_MODEL)
